# Initial kernel scaffold; baseline (speedup 1.0000x reference)
import jax
import jax.numpy as jnp
from jax.experimental import pallas as pl


def kernel(node_feats, node_attrs, edge_attrs, edge_feats, senders, receivers, params):
    raise NotImplementedError("write your pallas kernel here")


# speedup vs baseline: 3.5102x; 3.5102x over previous
import functools

import jax
import jax.numpy as jnp
from jax import lax
from jax.experimental import pallas as pl
from jax.experimental.pallas import tpu as pltpu
from jax.experimental.pallas import tpu_sc as plsc

NN = 10000
NE = 320000
D = 128
A = 16
F = 16
NLAYERS = 4

_NC = 2
_NS = 16
_NW = _NC * _NS
_EPW = NE // _NW
_CH = 80
_NCH = _EPW // _CH
NNP = 10240
_RPS = NNP // _NS

def _gather_body(nodes_hbm, ridx_hbm, sidx_hbm, out_in, out_out,
                 ridx_v, sidx_v, rbuf, sbuf, sem_r, sem_s):
    c = lax.axis_index("c")
    s = lax.axis_index("s")
    wid = s * _NC + c
    pltpu.sync_copy(ridx_hbm.at[wid], ridx_v)
    pltpu.sync_copy(sidx_hbm.at[wid], sidx_v)

    def body(j, _):
        cp_r = pltpu.async_copy(nodes_hbm.at[ridx_v.at[j]], rbuf, sem_r)
        cp_s = pltpu.async_copy(nodes_hbm.at[sidx_v.at[j]], sbuf, sem_s)
        cp_r.wait()
        cp_s.wait()
        row0 = wid * _EPW + j * _CH
        pltpu.sync_copy(rbuf, out_in.at[pl.ds(row0, _CH)])
        pltpu.sync_copy(sbuf, out_out.at[pl.ds(row0, _CH)])
        return ()

    lax.fori_loop(0, _NCH, body, (), unroll=False)


def _scatter_body(msg_hbm, ridx_hbm, zeros_hbm, out_agg, ridx_v, mbuf, acc, sem):
    c = lax.axis_index("c")
    s = lax.axis_index("s")
    wid = s * _NC + c
    pltpu.sync_copy(ridx_hbm.at[wid], ridx_v)
    r0 = s * _RPS
    pltpu.sync_copy(zeros_hbm.at[pl.ds(r0, _RPS)], acc.at[pl.ds(r0, _RPS)])
    plsc.subcore_barrier()

    def body(j, _):
        row0 = wid * _EPW + j * _CH
        cp = pltpu.async_copy(msg_hbm.at[pl.ds(row0, _CH)], mbuf, sem)
        cp.wait()
        pltpu.sync_copy(mbuf, acc.at[ridx_v.at[j]], add=True)
        return ()

    lax.fori_loop(0, _NCH, body, (), unroll=False)
    plsc.subcore_barrier()
    pltpu.sync_copy(acc.at[pl.ds(r0, _RPS)], out_agg.at[c, pl.ds(r0, _RPS)])


@functools.cache
def _build_sc_gather():
    mesh = plsc.VectorSubcoreMesh(core_axis_name="c", subcore_axis_name="s",
                                  num_cores=_NC)
    return pl.kernel(
        _gather_body,
        mesh=mesh,
        out_type=[
            jax.ShapeDtypeStruct((NE, D), jnp.float32),
            jax.ShapeDtypeStruct((NE, D), jnp.float32),
        ],
        scratch_types=[
            pltpu.VMEM((_NCH, _CH), jnp.int32),
            pltpu.VMEM((_NCH, _CH), jnp.int32),
            pltpu.VMEM((_CH, D), jnp.float32),
            pltpu.VMEM((_CH, D), jnp.float32),
            pltpu.SemaphoreType.DMA,
            pltpu.SemaphoreType.DMA,
        ],
    )


@functools.cache
def _build_sc_scatter():
    mesh = plsc.VectorSubcoreMesh(core_axis_name="c", subcore_axis_name="s",
                                  num_cores=_NC)
    return pl.kernel(
        _scatter_body,
        mesh=mesh,
        out_type=jax.ShapeDtypeStruct((_NC, NNP, D), jnp.float32),
        scratch_types=[
            pltpu.VMEM((_NCH, _CH), jnp.int32),
            pltpu.VMEM((_CH, D), jnp.float32),
            pltpu.VMEM_SHARED((NNP, D), jnp.float32),
            pltpu.SemaphoreType.DMA,
        ],
    )


def _sc_gather(nodes, ridx2, sidx2):
    return _build_sc_gather()(nodes, ridx2, sidx2)


def _sc_scatter(msg, ridx2, zeros_nd):
    return _build_sc_scatter()(msg, ridx2, zeros_nd)


_EB = 4000
_NB = 2000


def _embed_body(nf_ref, na_ref, w_ref, out_ref):
    out_ref[...] = (jnp.dot(nf_ref[...], w_ref[0:D, :])
                    + jnp.dot(na_ref[...], w_ref[D:D + A, :]))


def _embed(nf, na, w):
    return pl.pallas_call(
        _embed_body,
        grid=(NN // _NB,),
        in_specs=[
            pl.BlockSpec((_NB, D), lambda i: (i, 0)),
            pl.BlockSpec((_NB, A), lambda i: (i, 0)),
            pl.BlockSpec((D + A, D), lambda i: (0, 0)),
        ],
        out_specs=pl.BlockSpec((_NB, D), lambda i: (i, 0)),
        out_shape=jax.ShapeDtypeStruct((NN, D), jnp.float32),
    )(nf, na, w)


def _edge_mlp_body(inc_ref, out_ref, ef_ref, ea_ref, w0_ref, w1_ref, msg_ref):
    ea = ea_ref[...]
    h = (jnp.dot(inc_ref[...], w0_ref[0:D, :])
         + jnp.dot(out_ref[...], w0_ref[D:2 * D, :])
         + jnp.dot(ef_ref[...], w0_ref[2 * D:2 * D + F, :])
         + jnp.dot(ea, w0_ref[2 * D + F:2 * D + F + A, :]))
    m1 = h[:, :D] * jax.nn.sigmoid(h[:, D:])
    h2 = jnp.dot(m1, w1_ref[0:D, :]) + jnp.dot(ea, w1_ref[D:D + A, :])
    msg_ref[...] = h2[:, :D] * jax.nn.sigmoid(h2[:, D:])


def _edge_mlp(inc, outg, ef, ea, w0, w1):
    return pl.pallas_call(
        _edge_mlp_body,
        grid=(NE // _EB,),
        in_specs=[
            pl.BlockSpec((_EB, D), lambda i: (i, 0)),
            pl.BlockSpec((_EB, D), lambda i: (i, 0)),
            pl.BlockSpec((_EB, F), lambda i: (i, 0)),
            pl.BlockSpec((_EB, A), lambda i: (i, 0)),
            pl.BlockSpec((2 * D + F + A, 2 * D), lambda i: (0, 0)),
            pl.BlockSpec((D + A, 2 * D), lambda i: (0, 0)),
        ],
        out_specs=pl.BlockSpec((_EB, D), lambda i: (i, 0)),
        out_shape=jax.ShapeDtypeStruct((NE, D), jnp.float32),
    )(inc, outg, ef, ea, w0, w1)


def _node_update_body(nodes_ref, agg_ref, na_ref, w0_ref, w1_ref, out_ref):
    nodes = nodes_ref[...]
    na = na_ref[...]
    agg = agg_ref[0] + agg_ref[1]
    h = (jnp.dot(nodes, w0_ref[0:D, :])
         + jnp.dot(agg, w0_ref[D:2 * D, :])
         + jnp.dot(na, w0_ref[2 * D:2 * D + A, :]))
    t = h[:, :D] * jax.nn.sigmoid(h[:, D:])
    out_ref[...] = nodes + jnp.dot(t, w1_ref[0:D, :]) + jnp.dot(na, w1_ref[D:D + A, :])


def _node_update(nodes, agg2, na, w0, w1):
    return pl.pallas_call(
        _node_update_body,
        grid=(NN // _NB,),
        in_specs=[
            pl.BlockSpec((_NB, D), lambda i: (i, 0)),
            pl.BlockSpec((_NC, _NB, D), lambda i: (0, i, 0)),
            pl.BlockSpec((_NB, A), lambda i: (i, 0)),
            pl.BlockSpec((2 * D + A, 2 * D), lambda i: (0, 0)),
            pl.BlockSpec((D + A, D), lambda i: (0, 0)),
        ],
        out_specs=pl.BlockSpec((_NB, D), lambda i: (i, 0)),
        out_shape=jax.ShapeDtypeStruct((NN, D), jnp.float32),
    )(nodes, agg2, na, w0, w1)


def _decoder_body(nodes_ref, na_ref, wpre_ref, wpool_ref, wpost_ref, wout_ref,
                  out_ref, acc_ref):
    i = pl.program_id(0)

    @pl.when(i == 0)
    def _init():
        acc_ref[...] = jnp.zeros_like(acc_ref)

    na = na_ref[...]
    h = jnp.dot(nodes_ref[...], wpre_ref[0:D, :]) + jnp.dot(na, wpre_ref[D:D + A, :])
    t = h[:, :D] * jax.nn.sigmoid(h[:, D:])
    y = jnp.dot(t, wpool_ref[0:D, :]) + jnp.dot(na, wpool_ref[D:D + A, :])
    acc_ref[...] += jnp.sum(y, axis=0, keepdims=True)

    @pl.when(i == pl.num_programs(0) - 1)
    def _final():
        pooled = acc_ref[...] / NN
        h2 = jnp.dot(pooled, wpost_ref[...])
        t2 = h2[:, :D] * jax.nn.sigmoid(h2[:, D:])
        out_ref[...] = jnp.dot(t2, wout_ref[...],
                               precision=jax.lax.Precision.HIGHEST)


def _decoder(nodes, na, wpre, wpool, wpost, wout):
    return pl.pallas_call(
        _decoder_body,
        grid=(NN // _NB,),
        in_specs=[
            pl.BlockSpec((_NB, D), lambda i: (i, 0)),
            pl.BlockSpec((_NB, A), lambda i: (i, 0)),
            pl.BlockSpec((D + A, 2 * D), lambda i: (0, 0)),
            pl.BlockSpec((D + A, D), lambda i: (0, 0)),
            pl.BlockSpec((D, 2 * D), lambda i: (0, 0)),
            pl.BlockSpec((D, 1), lambda i: (0, 0)),
        ],
        out_specs=pl.BlockSpec((1, 1), lambda i: (0, 0)),
        out_shape=jax.ShapeDtypeStruct((1, 1), jnp.float32),
        scratch_shapes=[pltpu.VMEM((1, D), jnp.float32)],
    )(nodes, na, wpre, wpool, wpost, wout)


def kernel(node_feats, node_attrs, edge_attrs, edge_feats, senders, receivers,
           params):
    p = params
    ridx2 = receivers.reshape(_NW, _NCH, _CH)
    sidx2 = senders.reshape(_NW, _NCH, _CH)
    zeros_nd = jnp.zeros((NNP, D), jnp.float32)

    nodes = _embed(node_feats, node_attrs, p['embed'])
    for l in range(NLAYERS):
        inc, outg = _sc_gather(nodes, ridx2, sidx2)
        msg = _edge_mlp(inc, outg, edge_feats, edge_attrs,
                        p['l%d_msg0' % l], p['l%d_msg1' % l])
        agg2 = _sc_scatter(msg, ridx2, zeros_nd)
        nodes = _node_update(nodes, agg2, node_attrs,
                             p['l%d_upd0' % l], p['l%d_upd1' % l])
    out = _decoder(nodes, node_attrs, p['dec_pre0'], p['dec_prepool'],
                   p['dec_post0'], p['dec_out'])
    return jnp.squeeze(out)
